# baseline (speedup 1.0000x reference)
import jax
import jax.numpy as jnp
from jax import lax
from jax.experimental import pallas as pl
from jax.experimental.pallas import tpu as pltpu
from jax.experimental.pallas import tpu_sc as plsc

B = 16384
D = 32
PACK = 128 // D
NB = 1000000 // PACK
NC = 2
NS = 16
L = 16
NW = NC * NS
BPW = B // NW
CH = 128
NCH = BPW // CH
NG = CH // L


def _fpmc_body(uid_hbm, lic_hbm, nit_hbm, ui_hbm, iu_hbm, li_hbm, il_hbm,
               out_hbm, idx_u, idx_l, idx_n, q_u, q_l, q_n,
               g_ui, g_iu, g_li, g_il, out_v, sem):
    wid = lax.axis_index("s") * NC + lax.axis_index("c")
    base = wid * BPW

    pltpu.sync_copy(uid_hbm.at[pl.ds(base, BPW)], idx_u)
    pltpu.sync_copy(lic_hbm.at[pl.ds(base, BPW)], idx_l)
    pltpu.sync_copy(nit_hbm.at[pl.ds(base, BPW)], idx_n)

    def shift(i, carry):
        s = i * L
        idx = pl.ds(s, L)
        q_u[idx] = lax.shift_right_logical(idx_u[idx], 2)
        q_l[idx] = lax.shift_right_logical(idx_l[idx], 2)
        q_n[idx] = lax.shift_right_logical(idx_n[idx], 2)
        return carry

    lax.fori_loop(0, BPW // L, shift, 0)

    lanes = lax.iota(jnp.int32, L)

    def batch(c, carry):
        b0 = c * CH
        copies = [
            pltpu.async_copy(ui_hbm.at[q_u.at[pl.ds(b0, CH)]], g_ui, sem),
            pltpu.async_copy(iu_hbm.at[q_n.at[pl.ds(b0, CH)]], g_iu, sem),
            pltpu.async_copy(li_hbm.at[q_l.at[pl.ds(b0, CH)]], g_li, sem),
            pltpu.async_copy(il_hbm.at[q_n.at[pl.ds(b0, CH)]], g_il, sem),
        ]
        for cp in copies:
            cp.wait()

        def group(g, carry2):
            rows = g * L + lanes
            off_u = (idx_u[pl.ds(b0 + g * L, L)] & (PACK - 1)) * D
            off_l = (idx_l[pl.ds(b0 + g * L, L)] & (PACK - 1)) * D
            off_n = (idx_n[pl.ds(b0 + g * L, L)] & (PACK - 1)) * D
            acc = jnp.zeros((L,), jnp.float32)
            for d in range(D):
                acc = acc + (plsc.load_gather(g_ui, [rows, off_u + d]) *
                             plsc.load_gather(g_iu, [rows, off_n + d]))
                acc = acc + (plsc.load_gather(g_li, [rows, off_l + d]) *
                             plsc.load_gather(g_il, [rows, off_n + d]))
            out_v[pl.ds(b0 + g * L, L)] = 1.0 / (1.0 + jnp.exp(-acc))
            return carry2

        lax.fori_loop(0, NG, group, 0)
        return carry

    lax.fori_loop(0, NCH, batch, 0)
    pltpu.sync_copy(out_v, out_hbm.at[pl.ds(base, BPW)])


@jax.jit
def _fpmc(uid, lic, nit, UI, IU, LI, IL):
    fn = pl.kernel(
        _fpmc_body,
        out_type=jax.ShapeDtypeStruct((B,), jnp.float32),
        mesh=plsc.VectorSubcoreMesh(core_axis_name="c", subcore_axis_name="s",
                                    num_cores=NC, num_subcores=NS),
        scratch_types=[
            pltpu.VMEM((BPW,), jnp.int32),
            pltpu.VMEM((BPW,), jnp.int32),
            pltpu.VMEM((BPW,), jnp.int32),
            pltpu.VMEM((BPW,), jnp.int32),
            pltpu.VMEM((BPW,), jnp.int32),
            pltpu.VMEM((BPW,), jnp.int32),
            pltpu.VMEM((CH, 128), jnp.float32),
            pltpu.VMEM((CH, 128), jnp.float32),
            pltpu.VMEM((CH, 128), jnp.float32),
            pltpu.VMEM((CH, 128), jnp.float32),
            pltpu.VMEM((BPW,), jnp.float32),
            pltpu.SemaphoreType.DMA,
        ],
        compiler_params=pltpu.CompilerParams(use_tc_tiling_on_sc=False,
                                             needs_layout_passes=False),
    )
    return fn(uid, lic, nit, UI, IU, LI, IL)


def kernel(user_id, item_last_click, next_item, UI, IU, LI, IL):
    uid = user_id.reshape(-1).astype(jnp.int32)
    lic = item_last_click.reshape(-1).astype(jnp.int32)
    nit = next_item.reshape(-1).astype(jnp.int32)
    return _fpmc(uid, lic, nit,
                 UI.reshape(NB, 128), IU.reshape(NB, 128),
                 LI.reshape(NB, 128), IL.reshape(NB, 128))


# speedup vs baseline: 1.0014x; 1.0014x over previous
import jax
import jax.numpy as jnp
from jax import lax
from jax.experimental import pallas as pl
from jax.experimental.pallas import tpu as pltpu
from jax.experimental.pallas import tpu_sc as plsc

B = 16384
D = 32
PACK = 128 // D
NB = 1000000 // PACK
NC = 2
NS = 16
L = 16
NW = NC * NS
BPW = B // NW
CH = 128
NCH = BPW // CH
NG = CH // L


def _fpmc_body(uid_hbm, lic_hbm, nit_hbm, ui_hbm, iu_hbm, li_hbm, il_hbm,
               out_hbm, idx_u, idx_l, idx_n, q_u, q_l, q_n,
               g_ui, g_iu, g_li, g_il, out_v, sem):
    wid = lax.axis_index("s") * NC + lax.axis_index("c")
    base = wid * BPW

    pltpu.sync_copy(uid_hbm.at[pl.ds(base, BPW)], idx_u)
    pltpu.sync_copy(lic_hbm.at[pl.ds(base, BPW)], idx_l)
    pltpu.sync_copy(nit_hbm.at[pl.ds(base, BPW)], idx_n)

    def shift(i, carry):
        s = i * L
        idx = pl.ds(s, L)
        q_u[idx] = lax.shift_right_logical(idx_u[idx], 2)
        q_l[idx] = lax.shift_right_logical(idx_l[idx], 2)
        q_n[idx] = lax.shift_right_logical(idx_n[idx], 2)
        return carry

    lax.fori_loop(0, BPW // L, shift, 0)

    lanes = lax.iota(jnp.int32, L)

    def batch(c, carry):
        b0 = c * CH
        copies = [
            pltpu.async_copy(ui_hbm.at[q_u.at[pl.ds(b0, CH)]], g_ui, sem),
            pltpu.async_copy(iu_hbm.at[q_n.at[pl.ds(b0, CH)]], g_iu, sem),
            pltpu.async_copy(li_hbm.at[q_l.at[pl.ds(b0, CH)]], g_li, sem),
            pltpu.async_copy(il_hbm.at[q_n.at[pl.ds(b0, CH)]], g_il, sem),
        ]
        for cp in copies:
            cp.wait()

        def group(g, carry2):
            rows = g * L + lanes
            off_u = (idx_u[pl.ds(b0 + g * L, L)] & (PACK - 1)) * D
            off_l = (idx_l[pl.ds(b0 + g * L, L)] & (PACK - 1)) * D
            off_n = (idx_n[pl.ds(b0 + g * L, L)] & (PACK - 1)) * D
            acc = jnp.zeros((L,), jnp.float32)
            for d in range(D):
                acc = acc + (plsc.load_gather(g_ui, [rows, off_u + d]) *
                             plsc.load_gather(g_iu, [rows, off_n + d]))
                acc = acc + (plsc.load_gather(g_li, [rows, off_l + d]) *
                             plsc.load_gather(g_il, [rows, off_n + d]))
            out_v[pl.ds(b0 + g * L, L)] = 1.0 / (1.0 + jnp.exp(-acc))
            return carry2

        lax.fori_loop(0, NG, group, 0)
        return carry

    lax.fori_loop(0, NCH, batch, 0)
    pltpu.sync_copy(out_v, out_hbm.at[pl.ds(base, BPW)])


@jax.jit
def _fpmc(uid, lic, nit, UI, IU, LI, IL):
    fn = pl.kernel(
        _fpmc_body,
        out_type=jax.ShapeDtypeStruct((B,), jnp.float32),
        mesh=plsc.VectorSubcoreMesh(core_axis_name="c", subcore_axis_name="s",
                                    num_cores=NC, num_subcores=NS),
        scratch_types=[
            pltpu.VMEM((BPW,), jnp.int32),
            pltpu.VMEM((BPW,), jnp.int32),
            pltpu.VMEM((BPW,), jnp.int32),
            pltpu.VMEM((BPW,), jnp.int32),
            pltpu.VMEM((BPW,), jnp.int32),
            pltpu.VMEM((BPW,), jnp.int32),
            pltpu.VMEM((CH, 128), jnp.float32),
            pltpu.VMEM((CH, 128), jnp.float32),
            pltpu.VMEM((CH, 128), jnp.float32),
            pltpu.VMEM((CH, 128), jnp.float32),
            pltpu.VMEM((BPW,), jnp.float32),
            pltpu.SemaphoreType.DMA,
        ],
        compiler_params=pltpu.CompilerParams(use_tc_tiling_on_sc=False,
                                             needs_layout_passes=False),
    )
    return fn(uid, lic, nit, UI, IU, LI, IL)


def kernel(user_id, item_last_click, next_item, UI, IU, LI, IL):
    uid = user_id.reshape(-1).astype(jnp.int32)
    lic = item_last_click.reshape(-1).astype(jnp.int32)
    nit = next_item.reshape(-1).astype(jnp.int32)
    one = ((uid[0] & 0) + 1).astype(jnp.float32)
    return _fpmc(uid, lic, nit,
                 (UI * one).reshape(NB, 128), (IU * one).reshape(NB, 128),
                 (LI * one).reshape(NB, 128), (IL * one).reshape(NB, 128))
